# baseline (speedup 1.0000x reference)
import jax
import jax.numpy as jnp
import numpy as np
from jax import lax
from jax.experimental import pallas as pl
from jax.experimental.pallas import tpu as pltpu
from jax.experimental.pallas import tpu_sc as plsc

_PERM = (
    1, 3, 5, 0, 2, 6, 7, 4,
    10, 8, 12, 13, 15, 14, 11, 9,
    21, 23, 16, 17, 20, 18, 22, 19,
    28, 29, 27, 26, 31, 30, 24, 25,
    38, 37, 33, 35, 34, 39, 36, 32,
    43, 41, 47, 42, 44, 40, 45, 46,
    49, 55, 54, 48, 53, 51, 52, 50,
    61, 56, 58, 63, 57, 59, 60, 62,
)

_B, _C, _T = 128, 64, 4000
_ROWS = _B * _C
_NC, _NS = 2, 16
_NW = _NC * _NS
_RPW = _ROWS // _NW
_REG = 8
_NCHUNK = _RPW // _REG
_L = 16
_COLS = _T // _L


def _shuffle_block(buf, lp):
    def col(jj, carry):
        sl = pl.ds(jj * _L, _L)
        vals = [buf[lp[i], sl] for i in range(_REG)]
        for i in range(_REG):
            buf[i, sl] = vals[i]
        return carry

    lax.fori_loop(0, _COLS, col, 0)


def _sc_body(x_hbm, out_hbm, b0, b1, b2, gsem, ssem):
    wid = lax.axis_index("s") * _NC + lax.axis_index("c")
    batch0 = wid * (_B // _NW)
    bufs = (b0, b1, b2)

    def gather(ci):
        b = batch0 + ci // _REG
        c0 = (ci % _REG) * _REG
        return pltpu.make_async_copy(
            x_hbm.at[b, pl.ds(c0, _REG), 0], bufs[ci % 3], gsem
        )

    def scatter(ci):
        b = batch0 + ci // _REG
        c0 = (ci % _REG) * _REG
        return pltpu.make_async_copy(
            bufs[ci % 3], out_hbm.at[b, pl.ds(c0, _REG), 0], ssem
        )

    gather(0).start()
    gather(1).start()
    for ci in range(_NCHUNK):
        g = ci % _REG * _REG
        lp = tuple(_PERM[g + i] - g for i in range(_REG))
        gather(ci).wait()
        _shuffle_block(bufs[ci % 3], lp)
        scatter(ci).start()
        if ci + 2 < _NCHUNK:
            if ci >= 1:
                scatter(ci - 1).wait()
            gather(ci + 2).start()
    scatter(_NCHUNK - 2).wait()
    scatter(_NCHUNK - 1).wait()


def kernel(data_tensor, domain_labels, aux_labels):
    del domain_labels, aux_labels
    mesh = plsc.VectorSubcoreMesh(core_axis_name="c", subcore_axis_name="s")
    run = pl.kernel(
        _sc_body,
        out_type=jax.ShapeDtypeStruct((_B, _C, 1, _T), jnp.float32),
        mesh=mesh,
        scratch_types=[
            pltpu.VMEM((_REG, _T), jnp.float32),
            pltpu.VMEM((_REG, _T), jnp.float32),
            pltpu.VMEM((_REG, _T), jnp.float32),
            pltpu.SemaphoreType.DMA,
            pltpu.SemaphoreType.DMA,
        ],
    )
    return run(data_tensor)


# speedup vs baseline: 1.0013x; 1.0013x over previous
import jax
import jax.numpy as jnp
import numpy as np
from jax import lax
from jax.experimental import pallas as pl
from jax.experimental.pallas import tpu as pltpu
from jax.experimental.pallas import tpu_sc as plsc

_PERM = (
    1, 3, 5, 0, 2, 6, 7, 4,
    10, 8, 12, 13, 15, 14, 11, 9,
    21, 23, 16, 17, 20, 18, 22, 19,
    28, 29, 27, 26, 31, 30, 24, 25,
    38, 37, 33, 35, 34, 39, 36, 32,
    43, 41, 47, 42, 44, 40, 45, 46,
    49, 55, 54, 48, 53, 51, 52, 50,
    61, 56, 58, 63, 57, 59, 60, 62,
)

_B, _C, _T = 128, 64, 4000
_ROWS = _B * _C
_NC, _NS = 2, 16
_NW = _NC * _NS
_RPW = _ROWS // _NW
_REG = 8
_NCHUNK = _RPW // _REG
_L = 16
_COLS = _T // _L


def _shuffle_block(buf, lp):
    def col(jj, carry):
        sl = pl.ds(jj * _L, _L)
        vals = [buf[lp[i], 0, sl] for i in range(_REG)]
        for i in range(_REG):
            buf[i, 0, sl] = vals[i]
        return carry

    lax.fori_loop(0, _COLS, col, 0)


def _sc_body(x_hbm, out_hbm, b0, b1, b2, gsem, ssem):
    wid = lax.axis_index("s") * _NC + lax.axis_index("c")
    batch0 = wid * (_B // _NW)
    bufs = (b0, b1, b2)

    def gather(ci):
        b = batch0 + ci // _REG
        c0 = (ci % _REG) * _REG
        return pltpu.make_async_copy(
            x_hbm.at[b, pl.ds(c0, _REG)], bufs[ci % 3], gsem
        )

    def scatter(ci):
        b = batch0 + ci // _REG
        c0 = (ci % _REG) * _REG
        return pltpu.make_async_copy(
            bufs[ci % 3], out_hbm.at[b, pl.ds(c0, _REG)], ssem
        )

    gather(0).start()
    gather(1).start()
    for ci in range(_NCHUNK):
        g = ci % _REG * _REG
        lp = tuple(_PERM[g + i] - g for i in range(_REG))
        gather(ci).wait()
        _shuffle_block(bufs[ci % 3], lp)
        scatter(ci).start()
        if ci + 2 < _NCHUNK:
            if ci >= 1:
                scatter(ci - 1).wait()
            gather(ci + 2).start()
    scatter(_NCHUNK - 2).wait()
    scatter(_NCHUNK - 1).wait()


def kernel(data_tensor, domain_labels, aux_labels):
    del domain_labels, aux_labels
    mesh = plsc.VectorSubcoreMesh(core_axis_name="c", subcore_axis_name="s")
    run = pl.kernel(
        _sc_body,
        out_type=jax.ShapeDtypeStruct((_B, _C, 1, _T), jnp.float32),
        mesh=mesh,
        scratch_types=[
            pltpu.VMEM((_REG, 1, _T), jnp.float32),
            pltpu.VMEM((_REG, 1, _T), jnp.float32),
            pltpu.VMEM((_REG, 1, _T), jnp.float32),
            pltpu.SemaphoreType.DMA,
            pltpu.SemaphoreType.DMA,
        ],
    )
    return run(data_tensor)


# speedup vs baseline: 2.3077x; 2.3047x over previous
import jax
import jax.numpy as jnp
import numpy as np
from jax import lax
from jax.experimental import pallas as pl
from jax.experimental.pallas import tpu as pltpu
from jax.experimental.pallas import tpu_sc as plsc

_PERM = (
    1, 3, 5, 0, 2, 6, 7, 4,
    10, 8, 12, 13, 15, 14, 11, 9,
    21, 23, 16, 17, 20, 18, 22, 19,
    28, 29, 27, 26, 31, 30, 24, 25,
    38, 37, 33, 35, 34, 39, 36, 32,
    43, 41, 47, 42, 44, 40, 45, 46,
    49, 55, 54, 48, 53, 51, 52, 50,
    61, 56, 58, 63, 57, 59, 60, 62,
)

_B, _C, _T = 128, 64, 4000
_ROWS = _B * _C
_NC, _NS = 2, 16
_NW = _NC * _NS
_RPW = _ROWS // _NW
_REG = 8
_NCHUNK = _RPW // _REG
_L = 16
_COLS = _T // _L


def _shuffle_block(buf, lp):
    def col(jj, carry):
        sl = pl.ds(jj * _L, _L)
        vals = [buf[lp[i], sl] for i in range(_REG)]
        for i in range(_REG):
            buf[i, sl] = vals[i]
        return carry

    lax.fori_loop(0, _COLS, col, 0)


def _sc_body(x_hbm, out_hbm, b0, b1, b2, gsem, ssem):
    wid = lax.axis_index("s") * _NC + lax.axis_index("c")
    base = wid * _RPW
    bufs = (b0, b1, b2)

    def gather(ci):
        return pltpu.make_async_copy(
            x_hbm.at[pl.ds(base + ci * _REG, _REG)], bufs[ci % 3], gsem
        )

    def scatter(ci):
        return pltpu.make_async_copy(
            bufs[ci % 3], out_hbm.at[pl.ds(base + ci * _REG, _REG)], ssem
        )

    gather(0).start()
    gather(1).start()
    for ci in range(_NCHUNK):
        g = ci % _REG * _REG
        lp = tuple(_PERM[g + i] - g for i in range(_REG))
        gather(ci).wait()
        _shuffle_block(bufs[ci % 3], lp)
        scatter(ci).start()
        if ci + 2 < _NCHUNK:
            if ci >= 1:
                scatter(ci - 1).wait()
            gather(ci + 2).start()
    scatter(_NCHUNK - 2).wait()
    scatter(_NCHUNK - 1).wait()


def kernel(data_tensor, domain_labels, aux_labels):
    del domain_labels, aux_labels
    x = data_tensor.reshape(_ROWS, _T)
    mesh = plsc.VectorSubcoreMesh(core_axis_name="c", subcore_axis_name="s")
    run = pl.kernel(
        _sc_body,
        out_type=jax.ShapeDtypeStruct((_ROWS, _T), jnp.float32),
        mesh=mesh,
        scratch_types=[
            pltpu.VMEM((_REG, _T), jnp.float32),
            pltpu.VMEM((_REG, _T), jnp.float32),
            pltpu.VMEM((_REG, _T), jnp.float32),
            pltpu.SemaphoreType.DMA,
            pltpu.SemaphoreType.DMA,
        ],
    )
    out = run(x)
    return out.reshape(_B, _C, 1, _T)
